# Initial kernel scaffold; baseline (speedup 1.0000x reference)
import jax
import jax.numpy as jnp
from jax.experimental import pallas as pl


def kernel(input_ids, embedding, position_embedding):
    raise NotImplementedError("write your pallas kernel here")


# speedup vs baseline: 1.3481x; 1.3481x over previous
import functools

import jax
import jax.numpy as jnp
from jax import lax
from jax.experimental import pallas as pl
from jax.experimental.pallas import tpu as pltpu
from jax.experimental.pallas import tpu_sc as plsc

_CHUNK = 128


@functools.lru_cache(maxsize=None)
def _build(total_rows: int, seq_len: int, hidden: int):
    info = plsc.get_sparse_core_info()
    num_workers = info.num_cores * info.num_subcores
    rows_per_worker = total_rows // num_workers
    n_chunks = rows_per_worker // _CHUNK
    assert rows_per_worker * num_workers == total_rows
    assert n_chunks * _CHUNK == rows_per_worker
    assert seq_len % rows_per_worker == 0

    mesh = plsc.VectorSubcoreMesh(core_axis_name="c", subcore_axis_name="s")

    @functools.partial(
        pl.kernel,
        out_type=jax.ShapeDtypeStruct((total_rows, hidden), jnp.float32),
        mesh=mesh,
        scratch_types=[
            pltpu.VMEM((n_chunks, _CHUNK), jnp.int32),
            pltpu.VMEM((rows_per_worker, hidden), jnp.float32),
            pltpu.SemaphoreType.DMA,
        ],
    )
    def emb_kernel(ids_hbm, table_hbm, pos_hbm, out_hbm, idx_v, rows_v, sem):
        wid = lax.axis_index("s") * info.num_cores + lax.axis_index("c")
        base = wid * rows_per_worker
        pos_base = lax.rem(base, seq_len)
        pltpu.sync_copy(ids_hbm.at[pl.ds(wid * n_chunks, n_chunks)], idx_v)
        pltpu.sync_copy(pos_hbm.at[pl.ds(pos_base, rows_per_worker)], rows_v)
        copies = [
            pltpu.async_copy(
                table_hbm.at[idx_v.at[j]],
                rows_v.at[pl.ds(j * _CHUNK, _CHUNK)],
                sem,
                add=True,
            )
            for j in range(n_chunks)
        ]
        for cp in copies:
            cp.wait()
        pltpu.sync_copy(rows_v, out_hbm.at[pl.ds(base, rows_per_worker)])

    return emb_kernel


def kernel(input_ids, embedding, position_embedding):
    batch, seq_len = input_ids.shape
    hidden = embedding.shape[1]
    ids = input_ids.astype(jnp.int32).reshape(-1, _CHUNK)
    fn = _build(batch * seq_len, seq_len, hidden)
    out = fn(ids, embedding, position_embedding)
    return out.reshape(batch, seq_len, hidden)
